# Initial kernel scaffold; baseline (speedup 1.0000x reference)
import jax
import jax.numpy as jnp
from jax.experimental import pallas as pl


def kernel(x, centers):
    raise NotImplementedError("write your pallas kernel here")


# speedup vs baseline: 10.3818x; 10.3818x over previous
import functools

import jax
import jax.numpy as jnp
from jax.experimental import pallas as pl
from jax.experimental.pallas import tpu as pltpu

N, D, K = 4096, 64, 512
BLK = 512


def _body(x_ref, c_ref, out_idx_ref, out_min_ref, colmin_ref, colidx_ref):
    i = pl.program_id(0)
    xb = x_ref[...]
    c = c_ref[...]
    g = jax.lax.dot_general(
        xb, c, (((1,), (1,)), ((), ())),
        preferred_element_type=jnp.float32,
        precision=jax.lax.Precision.HIGHEST,
    )
    xn = jnp.sum(xb * xb, axis=1)
    cn = jnp.sum(c * c, axis=1)
    d2 = jnp.maximum(xn[:, None] + cn[None, :] - 2.0 * g, 0.0)

    out_min_ref[...] = jnp.sqrt(jnp.min(d2, axis=1))

    bmin = jnp.min(d2, axis=0)
    rows = jax.lax.broadcasted_iota(jnp.int32, d2.shape, 0) + i * BLK
    bidx = jnp.min(jnp.where(d2 == bmin[None, :], rows, jnp.int32(N)), axis=0)

    @pl.when(i == 0)
    def _():
        colmin_ref[...] = bmin
        colidx_ref[...] = bidx

    @pl.when(i > 0)
    def _():
        prev = colmin_ref[...]
        pidx = colidx_ref[...]
        upd = bmin < prev
        colmin_ref[...] = jnp.where(upd, bmin, prev)
        colidx_ref[...] = jnp.where(upd, bidx, pidx)

    @pl.when(i == pl.num_programs(0) - 1)
    def _():
        out_idx_ref[...] = colidx_ref[...]


@functools.partial(jax.jit, static_argnames=("interpret",))
def _run(x, centers, interpret=False):
    out_idx, out_min = pl.pallas_call(
        _body,
        grid=(N // BLK,),
        in_specs=[
            pl.BlockSpec((BLK, D), lambda i: (i, 0)),
            pl.BlockSpec((K, D), lambda i: (0, 0)),
        ],
        out_specs=[
            pl.BlockSpec((K,), lambda i: (0,)),
            pl.BlockSpec((BLK,), lambda i: (i,)),
        ],
        out_shape=[
            jax.ShapeDtypeStruct((K,), jnp.int32),
            jax.ShapeDtypeStruct((N,), jnp.float32),
        ],
        scratch_shapes=[
            pltpu.VMEM((K,), jnp.float32),
            pltpu.VMEM((K,), jnp.int32),
        ],
        compiler_params=pltpu.CompilerParams(
            dimension_semantics=("arbitrary",),
        ),
        interpret=interpret,
    )(x, centers)
    return out_idx, out_min


def kernel(x, centers):
    out_idx, out_min = _run(x, centers)
    return out_idx, out_min, centers
